# baseline (speedup 1.0000x reference)
import functools

import jax
import jax.numpy as jnp
from jax.experimental import pallas as pl
from jax.experimental.pallas import tpu as pltpu

_B = 2
_CIN = 256
_FIL = 256
_A = 9
_H = 48
_W = 48
_HW = _H * _W
_N = _HW * _A
_ROWS = _N // 128
_MAX_OUT = 300
_OUT_ROWS = 304
_NMS_T = 0.7


def _conv_head_body(xsh_ref, w1_ref, b1_ref, wh_ref, bh_ref, out_ref):
    acc = None
    for kh in range(3):
        for kw in range(3):
            xs = xsh_ref[0, kw, kh:kh + _H, :, :].reshape(_HW, _CIN)
            t = kh * 3 + kw
            p = jax.lax.dot_general(
                xs, w1_ref[t * _CIN:(t + 1) * _CIN, :], (((1,), (0,)), ((), ())),
                preferred_element_type=jnp.float32)
            acc = p if acc is None else acc + p
    l = jax.nn.relu(acc + b1_ref[...][None, :])
    out_ref[0] = jax.lax.dot_general(
        l, wh_ref[...], (((1,), (0,)), ((), ())),
        preferred_element_type=jnp.float32) + bh_ref[...][None, :]


def _nms_body(c0_ref, c1_ref, dx_ref, dy_ref, dw_ref, dh_ref,
              ax1_ref, ay1_ref, ax2_ref, ay2_ref, out_ref,
              x1_s, y1_s, x2_s, y2_s, ar_s, cur_s,
              kx1_s, ky1_s, kx2_s, ky2_s, kar_s):
    ax1 = ax1_ref[0]; ay1 = ay1_ref[0]; ax2 = ax2_ref[0]; ay2 = ay2_ref[0]
    widths = ax2 - ax1 + 1.0
    heights = ay2 - ay1 + 1.0
    cx = ax1 + 0.5 * widths
    cy = ay1 + 0.5 * heights
    dx = dx_ref[0]; dy = dy_ref[0]; dw = dw_ref[0]; dh = dh_ref[0]
    pcx = dx * widths + cx
    pcy = dy * heights + cy
    pw = jnp.exp(dw) * widths
    ph = jnp.exp(dh) * heights
    x1 = pcx - 0.5 * pw
    y1 = pcy - 0.5 * ph
    x2 = pcx + 0.5 * pw
    y2 = pcy + 0.5 * ph
    x1_s[...] = x1
    y1_s[...] = y1
    x2_s[...] = x2
    y2_s[...] = y2
    ar_s[...] = (x2 - x1) * (y2 - y1)

    c0 = c0_ref[0]; c1 = c1_ref[0]
    m = jnp.maximum(c0, c1)
    e0 = jnp.exp(c0 - m)
    e1 = jnp.exp(c1 - m)
    s = e0 + e1
    sm0 = e0 / s
    sm1 = e1 / s
    sc = jnp.maximum(sm0, sm1)
    cur_s[...] = jnp.where(sm1 > sm0, sc, -jnp.inf)

    lane = jax.lax.broadcasted_iota(jnp.int32, (_OUT_ROWS, 128), 1)
    out_ref[0] = jnp.where(lane == 4, jnp.float32(-1.0), jnp.float32(0.0))
    big = jnp.full((8, 128), 1e30, jnp.float32)
    kx1_s[...] = big
    ky1_s[...] = big
    kx2_s[...] = big
    ky2_s[...] = big
    kar_s[...] = big

    iota_flat = (jax.lax.broadcasted_iota(jnp.int32, (_ROWS, 128), 0) * 128
                 + jax.lax.broadcasted_iota(jnp.int32, (_ROWS, 128), 1))
    lane_row = jax.lax.broadcasted_iota(jnp.int32, (1, 128), 1)
    slot8 = (jax.lax.broadcasted_iota(jnp.int32, (8, 128), 0) * 128
             + jax.lax.broadcasted_iota(jnp.int32, (8, 128), 1))
    ninf = jnp.float32(-jnp.inf)

    def cond(st):
        k, go = st
        return go & (k < _MAX_OUT)

    def body(st):
        k, _ = st
        cur = cur_s[...]
        gmax = jnp.max(cur)
        active = gmax > ninf
        enc = jnp.where(cur == gmax, iota_flat, jnp.int32(2**31 - 1))
        idx = jnp.min(enc)
        idx = jnp.where(active, idx, 0)
        r = jax.lax.shift_right_logical(idx, 7)
        c = jax.lax.bitwise_and(idx, 127)

        def pick(ref):
            rowv = ref[pl.ds(r, 1), :]
            return jnp.max(jnp.where(lane_row == c, rowv, ninf))

        x1i = pick(x1_s); y1i = pick(y1_s); x2i = pick(x2_s); y2i = pick(y2_s)
        ari = pick(ar_s)

        kx1 = kx1_s[...]; ky1 = ky1_s[...]; kx2 = kx2_s[...]; ky2 = ky2_s[...]
        kar = kar_s[...]
        xx1 = jnp.maximum(kx1, x1i)
        yy1 = jnp.maximum(ky1, y1i)
        xx2 = jnp.minimum(kx2, x2i)
        yy2 = jnp.minimum(ky2, y2i)
        w = jnp.maximum(jnp.float32(0.0), xx2 - xx1)
        h = jnp.maximum(jnp.float32(0.0), yy2 - yy1)
        inter = w * h
        iou = inter / (kar + ari - inter + jnp.float32(1e-12))
        supp = jnp.max(iou) > jnp.float32(_NMS_T)

        @pl.when(active)
        def _():
            rowv = cur_s[pl.ds(r, 1), :]
            cur_s[pl.ds(r, 1), :] = jnp.where(lane_row == c, ninf, rowv)

        keep = active & jnp.logical_not(supp)

        @pl.when(keep)
        def _():
            kx1_s[...] = jnp.where(slot8 == k, x1i, kx1)
            ky1_s[...] = jnp.where(slot8 == k, y1i, ky1)
            kx2_s[...] = jnp.where(slot8 == k, x2i, kx2)
            ky2_s[...] = jnp.where(slot8 == k, y2i, ky2)
            kar_s[...] = jnp.where(slot8 == k, ari, kar)
            row = jnp.where(lane_row == 0, x1i,
                  jnp.where(lane_row == 1, y1i,
                  jnp.where(lane_row == 2, x2i,
                  jnp.where(lane_row == 3, y2i,
                  jnp.where(lane_row == 4, gmax, jnp.float32(0.0))))))
            out_ref[0, pl.ds(k, 1), :] = row

        return (k + keep.astype(jnp.int32), active)

    jax.lax.while_loop(cond, body, (jnp.int32(0), jnp.bool_(True)))


@jax.jit
def kernel(feature, anchors, W1, b1, Wc, bc, Wr, br):
    f32 = jnp.float32
    feature = feature.astype(f32)

    xp = jnp.transpose(jnp.pad(feature, ((0, 0), (0, 0), (1, 1), (1, 1))),
                       (0, 2, 3, 1))
    xsh = jnp.stack([xp[:, :, kw:kw + _W, :] for kw in range(3)], axis=1)
    w1_mat = jnp.concatenate(
        [W1[:, :, kh, kw].T for kh in range(3) for kw in range(3)], axis=0)

    Wc2 = Wc[:, :, 0, 0]
    Wr2 = Wr[:, :, 0, 0]
    cols = []
    bvals = []
    for f in range(6):
        for a in range(_A):
            if f < 2:
                cols.append(Wc2[2 * a + f])
                bvals.append(bc[2 * a + f])
            else:
                cols.append(Wr2[4 * a + (f - 2)])
                bvals.append(br[4 * a + (f - 2)])
    wh = jnp.pad(jnp.stack(cols, axis=1), ((0, 0), (0, 128 - 54)))
    bh = jnp.pad(jnp.stack(bvals), (0, 128 - 54))

    head = pl.pallas_call(
        _conv_head_body,
        grid=(_B,),
        in_specs=[
            pl.BlockSpec((1, 3, 50, _W, _CIN), lambda b: (b, 0, 0, 0, 0)),
            pl.BlockSpec((9 * _CIN, _FIL), lambda b: (0, 0)),
            pl.BlockSpec((_FIL,), lambda b: (0,)),
            pl.BlockSpec((_FIL, 128), lambda b: (0, 0)),
            pl.BlockSpec((128,), lambda b: (0,)),
        ],
        out_specs=pl.BlockSpec((1, _HW, 128), lambda b: (b, 0, 0)),
        out_shape=jax.ShapeDtypeStruct((_B, _HW, 128), f32),
    )(xsh, w1_mat, b1, wh, bh)

    def plane(f):
        return head[:, :, f * _A:(f + 1) * _A].reshape(_B, _ROWS, 128)

    c0, c1, dxp, dyp, dwp, dhp = (plane(f) for f in range(6))
    ap = [anchors[:, i].reshape(_ROWS, 128) for i in range(4)]

    vec_spec = pl.BlockSpec((1, _ROWS, 128), lambda b: (b, 0, 0))
    anc_spec = pl.BlockSpec((1, _ROWS, 128), lambda b: (0, 0, 0))
    scratch = ([pltpu.VMEM((_ROWS, 128), f32) for _ in range(6)]
               + [pltpu.VMEM((8, 128), f32) for _ in range(5)])

    out = pl.pallas_call(
        _nms_body,
        grid=(_B,),
        in_specs=[vec_spec] * 6 + [anc_spec] * 4,
        out_specs=pl.BlockSpec((1, _OUT_ROWS, 128), lambda b: (b, 0, 0)),
        out_shape=jax.ShapeDtypeStruct((_B, _OUT_ROWS, 128), f32),
        scratch_shapes=scratch,
    )(c0, c1, dxp, dyp, dwp, dhp,
      ap[0][None], ap[1][None], ap[2][None], ap[3][None])

    prop_b = out[:, :_MAX_OUT, 0:4]
    prop_s = out[:, :_MAX_OUT, 4]
    return (prop_b, prop_s)


# speedup vs baseline: 1.3098x; 1.3098x over previous
import functools

import jax
import jax.numpy as jnp
from jax.experimental import pallas as pl
from jax.experimental.pallas import tpu as pltpu

_B = 2
_CIN = 256
_FIL = 256
_A = 9
_H = 48
_W = 48
_HW = _H * _W
_N = _HW * _A
_ROWS = _N // 128
_MAX_OUT = 300
_OUT_ROWS = 304
_NMS_T = 0.7


def _conv_head_body(xsh_ref, w1_ref, b1_ref, wh_ref, bh_ref, out_ref):
    acc = None
    for kh in range(3):
        for kw in range(3):
            xs = xsh_ref[0, kw, kh:kh + _H, :, :].reshape(_HW, _CIN)
            t = kh * 3 + kw
            p = jax.lax.dot_general(
                xs, w1_ref[t * _CIN:(t + 1) * _CIN, :], (((1,), (0,)), ((), ())),
                preferred_element_type=jnp.float32)
            acc = p if acc is None else acc + p
    l = jax.nn.relu(acc + b1_ref[...][None, :])
    out_ref[0] = jax.lax.dot_general(
        l, wh_ref[...], (((1,), (0,)), ((), ())),
        preferred_element_type=jnp.float32) + bh_ref[...][None, :]


def _nms_body(c0_ref, c1_ref, dx_ref, dy_ref, dw_ref, dh_ref,
              ax1_ref, ay1_ref, ax2_ref, ay2_ref, out_ref,
              x1_s, y1_s, x2_s, y2_s, ar_s, cur_s):
    ax1 = ax1_ref[0]; ay1 = ay1_ref[0]; ax2 = ax2_ref[0]; ay2 = ay2_ref[0]
    widths = ax2 - ax1 + 1.0
    heights = ay2 - ay1 + 1.0
    cx = ax1 + 0.5 * widths
    cy = ay1 + 0.5 * heights
    dx = dx_ref[0]; dy = dy_ref[0]; dw = dw_ref[0]; dh = dh_ref[0]
    pcx = dx * widths + cx
    pcy = dy * heights + cy
    pw = jnp.exp(dw) * widths
    ph = jnp.exp(dh) * heights
    x1 = pcx - 0.5 * pw
    y1 = pcy - 0.5 * ph
    x2 = pcx + 0.5 * pw
    y2 = pcy + 0.5 * ph
    x1_s[...] = x1
    y1_s[...] = y1
    x2_s[...] = x2
    y2_s[...] = y2
    ar_s[...] = (x2 - x1) * (y2 - y1)

    c0 = c0_ref[0]; c1 = c1_ref[0]
    m = jnp.maximum(c0, c1)
    e0 = jnp.exp(c0 - m)
    e1 = jnp.exp(c1 - m)
    s = e0 + e1
    sm0 = e0 / s
    sm1 = e1 / s
    sc = jnp.maximum(sm0, sm1)
    cur_s[...] = jnp.where(sm1 > sm0, sc, -jnp.inf)

    lane = jax.lax.broadcasted_iota(jnp.int32, (_OUT_ROWS, 128), 1)
    out_ref[0] = jnp.where(lane == 4, jnp.float32(-1.0), jnp.float32(0.0))

    iota_flat = (jax.lax.broadcasted_iota(jnp.int32, (_ROWS, 128), 0) * 128
                 + jax.lax.broadcasted_iota(jnp.int32, (_ROWS, 128), 1))
    lane_row = jax.lax.broadcasted_iota(jnp.int32, (1, 128), 1)
    ninf = jnp.float32(-jnp.inf)

    G = 4
    BIGI = jnp.int32(2**31 - 1)
    T = jnp.float32(_NMS_T)

    def cond(st):
        k, go = st
        return go & (k < _MAX_OUT)

    def body(st):
        k, _ = st
        cur = cur_s[...]

        vals, idxs, acts = [], [], []
        curv = cur
        for _j in range(G):
            mj = jnp.max(curv)
            ij = jnp.min(jnp.where(curv == mj, iota_flat, BIGI))
            aj = mj > ninf
            ij = jnp.where(aj, ij, 0)
            curv = jnp.where(iota_flat == ij, ninf, curv) if _j + 1 < G else curv
            vals.append(mj); idxs.append(ij); acts.append(aj)

        boxes = []
        for _j in range(G):
            r = jax.lax.shift_right_logical(idxs[_j], 7)
            c = jax.lax.bitwise_and(idxs[_j], 127)

            def pick(ref, r=r, c=c):
                rowv = ref[pl.ds(r, 1), :]
                return jnp.max(jnp.where(lane_row == c, rowv, ninf))

            boxes.append((pick(x1_s), pick(y1_s), pick(x2_s), pick(y2_s), pick(ar_s)))

        x1v = x1_s[...]; y1v = y1_s[...]; x2v = x2_s[...]; y2v = y2_s[...]
        arv = ar_s[...]
        sups = []
        for _j in range(G):
            x1i, y1i, x2i, y2i, ari = boxes[_j]
            xx1 = jnp.maximum(x1i, x1v)
            yy1 = jnp.maximum(y1i, y1v)
            xx2 = jnp.minimum(x2i, x2v)
            yy2 = jnp.minimum(y2i, y2v)
            w = jnp.maximum(jnp.float32(0.0), xx2 - xx1)
            h = jnp.maximum(jnp.float32(0.0), yy2 - yy1)
            inter = w * h
            iou = inter / (ari + arv - inter + jnp.float32(1e-12))
            sups.append(iou > T)

        def hit(j, kk):
            m = jnp.where(iota_flat == idxs[kk], sups[j].astype(jnp.int32), 0)
            return jnp.max(m) > 0

        accs = [acts[0]]
        for kk in range(1, G):
            s = hit(0, kk) & accs[0]
            for j in range(1, kk):
                s = s | (hit(j, kk) & accs[j])
            accs.append(acts[kk] & jnp.logical_not(s))

        supA = accs[0] & sups[0]
        for j in range(1, G):
            supA = supA | (accs[j] & sups[j])
        cur_s[...] = jnp.where(supA, ninf, cur)

        rowk = k
        for j in range(G):
            x1i, y1i, x2i, y2i, _ = boxes[j]
            row = jnp.where(lane_row == 0, x1i,
                  jnp.where(lane_row == 1, y1i,
                  jnp.where(lane_row == 2, x2i,
                  jnp.where(lane_row == 3, y2i,
                  jnp.where(lane_row == 4, vals[j], jnp.float32(0.0))))))
            aj = accs[j]

            @pl.when(aj)
            def _(rowk=rowk, row=row):
                out_ref[0, pl.ds(rowk, 1), :] = row

            rowk = rowk + aj.astype(jnp.int32)

        return (rowk, acts[0])

    jax.lax.while_loop(cond, body, (jnp.int32(0), jnp.bool_(True)))


@jax.jit
def kernel(feature, anchors, W1, b1, Wc, bc, Wr, br):
    f32 = jnp.float32
    feature = feature.astype(f32)

    xp = jnp.transpose(jnp.pad(feature, ((0, 0), (0, 0), (1, 1), (1, 1))),
                       (0, 2, 3, 1))
    xsh = jnp.stack([xp[:, :, kw:kw + _W, :] for kw in range(3)], axis=1)
    w1_mat = jnp.concatenate(
        [W1[:, :, kh, kw].T for kh in range(3) for kw in range(3)], axis=0)

    Wc2 = Wc[:, :, 0, 0]
    Wr2 = Wr[:, :, 0, 0]
    cols = []
    bvals = []
    for f in range(6):
        for a in range(_A):
            if f < 2:
                cols.append(Wc2[2 * a + f])
                bvals.append(bc[2 * a + f])
            else:
                cols.append(Wr2[4 * a + (f - 2)])
                bvals.append(br[4 * a + (f - 2)])
    wh = jnp.pad(jnp.stack(cols, axis=1), ((0, 0), (0, 128 - 54)))
    bh = jnp.pad(jnp.stack(bvals), (0, 128 - 54))

    head = pl.pallas_call(
        _conv_head_body,
        grid=(_B,),
        in_specs=[
            pl.BlockSpec((1, 3, 50, _W, _CIN), lambda b: (b, 0, 0, 0, 0)),
            pl.BlockSpec((9 * _CIN, _FIL), lambda b: (0, 0)),
            pl.BlockSpec((_FIL,), lambda b: (0,)),
            pl.BlockSpec((_FIL, 128), lambda b: (0, 0)),
            pl.BlockSpec((128,), lambda b: (0,)),
        ],
        out_specs=pl.BlockSpec((1, _HW, 128), lambda b: (b, 0, 0)),
        out_shape=jax.ShapeDtypeStruct((_B, _HW, 128), f32),
    )(xsh, w1_mat, b1, wh, bh)

    def plane(f):
        return head[:, :, f * _A:(f + 1) * _A].reshape(_B, _ROWS, 128)

    c0, c1, dxp, dyp, dwp, dhp = (plane(f) for f in range(6))
    ap = [anchors[:, i].reshape(_ROWS, 128) for i in range(4)]

    vec_spec = pl.BlockSpec((1, _ROWS, 128), lambda b: (b, 0, 0))
    anc_spec = pl.BlockSpec((1, _ROWS, 128), lambda b: (0, 0, 0))
    scratch = [pltpu.VMEM((_ROWS, 128), f32) for _ in range(6)]

    out = pl.pallas_call(
        _nms_body,
        grid=(_B,),
        in_specs=[vec_spec] * 6 + [anc_spec] * 4,
        out_specs=pl.BlockSpec((1, _OUT_ROWS, 128), lambda b: (b, 0, 0)),
        out_shape=jax.ShapeDtypeStruct((_B, _OUT_ROWS, 128), f32),
        scratch_shapes=scratch,
    )(c0, c1, dxp, dyp, dwp, dhp,
      ap[0][None], ap[1][None], ap[2][None], ap[3][None])

    prop_b = out[:, :_MAX_OUT, 0:4]
    prop_s = out[:, :_MAX_OUT, 4]
    return (prop_b, prop_s)
